# baseline (speedup 1.0000x reference)
import jax
import jax.numpy as jnp
from jax.experimental import pallas as pl
from jax.experimental.pallas import tpu as pltpu

S, H, I = 2048, 4096, 14336
I_BLK = 256
K_BLK = 2048
H_BLK = 1024
VMEM_LIMIT = 63 * 1024 * 1024

_NT = (((1,), (1,)), ((), ()))


R_CHUNK = 512
R_CHUNK1 = 512


def _gate_up_kernel(x_ref, gate_ref, up_ref, h_ref):
    gate_w = gate_ref[...]
    up_w = up_ref[...]
    for r in range(S // R_CHUNK1):
        rows = pl.ds(r * R_CHUNK1, R_CHUNK1)
        x = x_ref[rows, :]
        g = jax.lax.dot_general(x, gate_w, _NT,
                                preferred_element_type=jnp.float32)
        u = jax.lax.dot_general(x, up_w, _NT,
                                preferred_element_type=jnp.float32)
        h_ref[rows, :] = (g * jax.nn.sigmoid(g) * u).astype(jnp.bfloat16)


def _down_kernel(h_ref, down_ref, out_ref):
    k = pl.program_id(1)
    scale = (k != 0).astype(jnp.float32)
    down_w = down_ref[...].astype(jnp.bfloat16)
    for r in range(S // R_CHUNK):
        rows = pl.ds(r * R_CHUNK, R_CHUNK)
        h = h_ref[rows, :]
        contrib = jax.lax.dot_general(
            h, down_w, _NT, preferred_element_type=jnp.float32)
        out_ref[rows, :] = out_ref[rows, :] * scale + contrib


def kernel(x, gate_W, up_W, down_W):
    B = x.shape[0]
    xb = x.reshape(S, H)

    h = pl.pallas_call(
        _gate_up_kernel,
        grid=(I // I_BLK,),
        in_specs=[
            pl.BlockSpec((S, H), lambda i: (0, 0)),
            pl.BlockSpec((I_BLK, H), lambda i: (i, 0)),
            pl.BlockSpec((I_BLK, H), lambda i: (i, 0)),
        ],
        out_specs=pl.BlockSpec((S, I_BLK), lambda i: (0, i)),
        out_shape=jax.ShapeDtypeStruct((S, I), jnp.bfloat16),
        compiler_params=pltpu.CompilerParams(
            dimension_semantics=("arbitrary",),
            vmem_limit_bytes=VMEM_LIMIT,
        ),
    )(xb, gate_W, up_W)

    out = pl.pallas_call(
        _down_kernel,
        grid=(H // H_BLK, I // K_BLK),
        in_specs=[
            pl.BlockSpec((S, K_BLK), lambda hh, k: (0, k)),
            pl.BlockSpec((H_BLK, K_BLK), lambda hh, k: (hh, k)),
        ],
        out_specs=pl.BlockSpec((S, H_BLK), lambda hh, k: (0, hh)),
        out_shape=jax.ShapeDtypeStruct((S, H), jnp.float32),
        compiler_params=pltpu.CompilerParams(
            dimension_semantics=("parallel", "arbitrary"),
            vmem_limit_bytes=VMEM_LIMIT,
        ),
    )(h, down_W)
    return out.reshape(B, S, H)


# speedup vs baseline: 1.0034x; 1.0034x over previous
import jax
import jax.numpy as jnp
from jax.experimental import pallas as pl
from jax.experimental.pallas import tpu as pltpu

S, H, I = 2048, 4096, 14336
I_BLK = 256
K_BLK = 2048
H_BLK = 1024
VMEM_LIMIT = 63 * 1024 * 1024

_NT = (((1,), (1,)), ((), ()))


R_CHUNK = 1024
R_CHUNK1 = 256


def _gate_up_kernel(x_ref, gate_ref, up_ref, h_ref):
    gate_w = gate_ref[...]
    up_w = up_ref[...]
    for r in range(S // R_CHUNK1):
        rows = pl.ds(r * R_CHUNK1, R_CHUNK1)
        x = x_ref[rows, :]
        g = jax.lax.dot_general(x, gate_w, _NT,
                                preferred_element_type=jnp.float32)
        u = jax.lax.dot_general(x, up_w, _NT,
                                preferred_element_type=jnp.float32)
        h_ref[rows, :] = (g * jax.nn.sigmoid(g) * u).astype(jnp.bfloat16)


def _down_kernel(h_ref, down_ref, out_ref):
    k = pl.program_id(1)
    scale = (k != 0).astype(jnp.float32)
    down_w = down_ref[...].astype(jnp.bfloat16)
    for r in range(S // R_CHUNK):
        rows = pl.ds(r * R_CHUNK, R_CHUNK)
        h = h_ref[rows, :]
        contrib = jax.lax.dot_general(
            h, down_w, _NT, preferred_element_type=jnp.float32)
        out_ref[rows, :] = out_ref[rows, :] * scale + contrib


def kernel(x, gate_W, up_W, down_W):
    B = x.shape[0]
    xb = x.reshape(S, H)

    h = pl.pallas_call(
        _gate_up_kernel,
        grid=(I // I_BLK,),
        in_specs=[
            pl.BlockSpec((S, H), lambda i: (0, 0)),
            pl.BlockSpec((I_BLK, H), lambda i: (i, 0)),
            pl.BlockSpec((I_BLK, H), lambda i: (i, 0)),
        ],
        out_specs=pl.BlockSpec((S, I_BLK), lambda i: (0, i)),
        out_shape=jax.ShapeDtypeStruct((S, I), jnp.bfloat16),
        compiler_params=pltpu.CompilerParams(
            dimension_semantics=("arbitrary",),
            vmem_limit_bytes=VMEM_LIMIT,
        ),
    )(xb, gate_W, up_W)

    out = pl.pallas_call(
        _down_kernel,
        grid=(H // H_BLK, I // K_BLK),
        in_specs=[
            pl.BlockSpec((S, K_BLK), lambda hh, k: (0, k)),
            pl.BlockSpec((H_BLK, K_BLK), lambda hh, k: (hh, k)),
        ],
        out_specs=pl.BlockSpec((S, H_BLK), lambda hh, k: (0, hh)),
        out_shape=jax.ShapeDtypeStruct((S, H), jnp.float32),
        compiler_params=pltpu.CompilerParams(
            dimension_semantics=("parallel", "arbitrary"),
            vmem_limit_bytes=VMEM_LIMIT,
        ),
    )(h, down_W)
    return out.reshape(B, S, H)
